# Initial kernel scaffold; baseline (speedup 1.0000x reference)
import jax
import jax.numpy as jnp
from jax.experimental import pallas as pl


def kernel(x, edge_index, edge_attr, node_W, node_b, lin_W, lin_b, edge_W, edge_b, eps, bn_gamma, bn_beta, cross):
    raise NotImplementedError("write your pallas kernel here")


# speedup vs baseline: 1.8603x; 1.8603x over previous
import functools

import jax
import jax.numpy as jnp
from jax import lax
from jax.experimental import pallas as pl
from jax.experimental.pallas import tpu as pltpu
from jax.experimental.pallas import tpu_sc as plsc

NC = 2
NS = 16
LANES = 16


def _enc_body(x_ref, w_ref, b_ref, o_ref):
    o_ref[0] = (
        jnp.dot(x_ref[...], w_ref[0], preferred_element_type=jnp.float32)
        + b_ref[0]
    )


def _encoder(x, node_W, node_b, bn):
    n, d = x.shape
    nblk = pl.cdiv(n, bn)
    return pl.pallas_call(
        _enc_body,
        grid=(2, nblk),
        in_specs=[
            pl.BlockSpec((bn, d), lambda s, i: (i, 0)),
            pl.BlockSpec((1, d, d), lambda s, i: (s, 0, 0)),
            pl.BlockSpec((1, 1, d), lambda s, i: (s, 0, 0)),
        ],
        out_specs=pl.BlockSpec((1, bn, d), lambda s, i: (s, i, 0)),
        out_shape=jax.ShapeDtypeStruct((2, n, d), jnp.float32),
    )(x, node_W, node_b.reshape(2, 1, d))


def _eemb_body(a_ref, w_ref, b_ref, o_ref):
    o_ref[0] = (
        jnp.dot(a_ref[...], w_ref[0], preferred_element_type=jnp.float32)
        + b_ref[0]
    )


def _edge_embeddings(edge_attr, ew, eb, be):
    e, de = edge_attr.shape
    k, _, d = ew.shape
    nblk = pl.cdiv(e, be)
    return pl.pallas_call(
        _eemb_body,
        grid=(k, nblk),
        in_specs=[
            pl.BlockSpec((be, de), lambda j, i: (i, 0)),
            pl.BlockSpec((1, de, d), lambda j, i: (j, 0, 0)),
            pl.BlockSpec((1, 1, d), lambda j, i: (j, 0, 0)),
        ],
        out_specs=pl.BlockSpec((1, be, d), lambda j, i: (j, i, 0)),
        out_shape=jax.ShapeDtypeStruct((k, e, d), jnp.float32),
    )(edge_attr, ew, eb.reshape(k, 1, d))


def _make_upd_body(do_relu):
    def body(h_ref, g_ref, wh_ref, wa_ref, b_ref, m_ref, o_ref):
        ab = []
        for s in range(2):
            t = (
                jnp.dot(h_ref[s], wh_ref[s], preferred_element_type=jnp.float32)
                + jnp.dot(g_ref[s], wa_ref[s], preferred_element_type=jnp.float32)
                + b_ref[s]
            )
            if do_relu:
                t = jnp.maximum(t, 0.0)
            ab.append(t)
        o_ref[0] = m_ref[0, 0] * ab[0] + m_ref[0, 1] * ab[1]
        o_ref[1] = m_ref[1, 0] * ab[0] + m_ref[1, 1] * ab[1]
    return body


def _update(h, agg, wh, wa, bb, mm, do_relu, bn):
    _, n, d = h.shape
    nblk = pl.cdiv(n, bn)
    return pl.pallas_call(
        _make_upd_body(do_relu),
        grid=(nblk,),
        in_specs=[
            pl.BlockSpec((2, bn, d), lambda i: (0, i, 0)),
            pl.BlockSpec((2, bn, d), lambda i: (0, i, 0)),
            pl.BlockSpec((2, d, d), lambda i: (0, 0, 0)),
            pl.BlockSpec((2, d, d), lambda i: (0, 0, 0)),
            pl.BlockSpec((2, 1, d), lambda i: (0, 0, 0)),
            pl.BlockSpec(memory_space=pltpu.SMEM),
        ],
        out_specs=pl.BlockSpec((2, bn, d), lambda i: (0, i, 0)),
        out_shape=jax.ShapeDtypeStruct((2, n, d), jnp.float32),
    )(h, agg, wh, wa, bb.reshape(2, 1, d), mm)


def _make_sc_gin(n, e, d):
    cb = 128
    chunks = e // cb
    nps = n // NS
    zr = 125
    nz = nps // zr
    jpd = d // LANES

    def body(h_ref, e_ref, row_ref, col_ref, out_ref,
             row_v, col_v, h_v, e_v, zero_v, agg_sh, sem):
        cid = lax.axis_index("c")
        sid = lax.axis_index("s")

        def zbody(i, _):
            for j in range(jpd):
                zero_v[i, pl.ds(j * LANES, LANES)] = jnp.zeros(
                    (LANES,), jnp.float32)
            return 0
        lax.fori_loop(0, zr, zbody, 0)
        for t in range(nz):
            r0 = sid * nps + t * zr
            pltpu.sync_copy(zero_v, agg_sh.at[pl.ds(r0, zr)])
        plsc.subcore_barrier()

        full = chunks // NS
        extra = chunks % NS
        trip = full + jnp.where(sid < extra, 1, 0)

        def chunk_body(t, _):
            ch = sid + t * NS
            base = pl.multiple_of(ch * cb, cb)
            ebase = pl.multiple_of(cid * e + base, cb)
            pltpu.sync_copy(row_ref.at[pl.ds(ebase, cb)], row_v)
            gather = pltpu.async_copy(h_ref.at[row_v], h_v, sem)
            pltpu.sync_copy(col_ref.at[pl.ds(base, cb)], col_v)
            pltpu.sync_copy(e_ref.at[pl.ds(ebase, cb)], e_v)
            gather.wait()

            def edge_body(i, _):
                for j in range(jpd):
                    sl = pl.ds(j * LANES, LANES)
                    e_v[i, sl] = jnp.maximum(h_v[i, sl] + e_v[i, sl], 0.0)
                return 0
            lax.fori_loop(0, cb, edge_body, 0)

            pltpu.sync_copy(e_v, agg_sh.at[col_v], add=True)
            return 0
        lax.fori_loop(0, trip, chunk_body, 0)

        plsc.subcore_barrier()
        s8 = -(-n // (NS * 8)) * 8
        r0 = pl.multiple_of(sid * s8, 8)
        rows_last = n - s8 * (NS - 1)

        @pl.when(sid < NS - 1)
        def _():
            pltpu.sync_copy(agg_sh.at[pl.ds(r0, s8)],
                            out_ref.at[cid, pl.ds(r0, s8)])

        @pl.when(sid == NS - 1)
        def _():
            rl = pl.multiple_of((NS - 1) * s8, 8)
            pltpu.sync_copy(agg_sh.at[pl.ds(rl, rows_last)],
                            out_ref.at[cid, pl.ds(rl, rows_last)])

    return pl.kernel(
        body,
        out_type=jax.ShapeDtypeStruct((2, n, d), jnp.float32),
        mesh=plsc.VectorSubcoreMesh(core_axis_name="c", subcore_axis_name="s",
                                    num_cores=NC, num_subcores=NS),
        scratch_types=[
            pltpu.VMEM((cb,), jnp.int32),
            pltpu.VMEM((cb,), jnp.int32),
            pltpu.VMEM((cb, d), jnp.float32),
            pltpu.VMEM((cb, d), jnp.float32),
            pltpu.VMEM((zr, d), jnp.float32),
            pltpu.VMEM_SHARED((n, d), jnp.float32),
            pltpu.SemaphoreType.DMA,
        ],
    )


def kernel(x, edge_index, edge_attr, node_W, node_b, lin_W, lin_b,
           edge_W, edge_b, eps, bn_gamma, bn_beta, cross):
    n, d = x.shape
    e = edge_index.shape[1]
    nl = lin_W.shape[0]

    row = edge_index[0]
    col = edge_index[1]
    row_off = jnp.concatenate([row, row + n])

    bn_inv = 1.0 / jnp.sqrt(1.0 + 1e-5)
    g = bn_gamma * bn_inv
    wa = lin_W * g[:, :, None, :]
    wh = wa * (1.0 + eps)[:, :, None, None]
    bb = lin_b * g + bn_beta
    m00 = cross[:, 0, 0]
    m01 = cross[:, 0, 1]
    m10 = cross[:, 1, 0] * m00
    m11 = cross[:, 1, 0] * m01 + cross[:, 1, 1]
    mm = jnp.stack([jnp.stack([m00, m01], -1),
                    jnp.stack([m10, m11], -1)], 1)

    h = _encoder(x, node_W, node_b, bn=1000)
    e_all = _edge_embeddings(
        edge_attr, edge_W.reshape(nl * 2, -1, d),
        edge_b.reshape(nl * 2, d), be=2000)

    sc_gin = _make_sc_gin(n, e, d)
    for l in range(nl):
        e_l = e_all[2 * l:2 * l + 2].reshape(2 * e, d)
        agg = sc_gin(h.reshape(2 * n, d), e_l, row_off, col)
        h = _update(h, agg, wh[l], wa[l], bb[l], mm[l],
                    do_relu=(l < nl - 1), bn=1000)
    return (h[0], h[1])
